# Initial kernel scaffold; baseline (speedup 1.0000x reference)
import jax
import jax.numpy as jnp
from jax.experimental import pallas as pl


def kernel(x, edge_index, edge_type, edge_type_list, edge_weight, rel_emb, int_emb, rbf1_W, rbf1_b, rbf2_W, rbf2_b, ee_W, ee_b, root0_W, root0_b, root1_W, root1_b, root2_W, root2_b, root3_W, root3_b, cat_W, cat_b):
    raise NotImplementedError("write your pallas kernel here")


# speedup vs baseline: 1.0820x; 1.0820x over previous
import functools
import jax
import jax.numpy as jnp
from jax.experimental import pallas as pl
from jax.experimental.pallas import tpu as pltpu

N = 10000
E = 320000
H = 128
GAMMA = 10.0
NEG_SLOPE = 0.01

_EB = 3200


def _leaky(x):
    return jnp.where(x >= 0, x, NEG_SLOPE * x)


def _l2n(v):
    n = jnp.sqrt(jnp.sum(v * v, axis=-1, keepdims=True))
    return v / jnp.maximum(n, 1e-12)


def _edge_feat_body(ew_ref, etl_ref, relW1_ref, intW2_ref, m1_ref, m2_ref,
                    c0_ref, c1_ref, c2_ref, out_ref):
    w = ew_ref[...]
    t = etl_ref[...]
    eb = w.shape[0]
    cidx = jax.lax.broadcasted_iota(jnp.int32, (eb, 32), 1)
    cen = cidx.astype(jnp.float32) * 0.1
    rbf = jnp.exp(-GAMMA * (w - cen) ** 2)
    rbf = jnp.where(cidx < 20, rbf, 0.0)
    sca = jnp.dot(rbf, m1_ref[...], preferred_element_type=jnp.float32) + c1_ref[...]
    grp = jnp.dot(rbf, m2_ref[...], preferred_element_type=jnp.float32) + c2_ref[...]
    wi = jnp.clip(w.astype(jnp.int32), 0, 15)
    oh_i = (jax.lax.broadcasted_iota(jnp.int32, (eb, 16), 1) == wi).astype(jnp.float32)
    prop_attr = jnp.dot(oh_i, intW2_ref[...], preferred_element_type=jnp.float32) + c0_ref[...]
    oh_t = (jax.lax.broadcasted_iota(jnp.int32, (eb, 8), 1) == t).astype(jnp.float32)
    base = jnp.dot(oh_t, relW1_ref[...], preferred_element_type=jnp.float32)
    attr = jnp.where(t == 0, prop_attr, jnp.where(t == 1, sca, grp))
    xe = base + attr
    out_ref[...] = _leaky(_l2n(xe))


def _edge_features(edge_weight, edge_type_list, rel_emb, int_emb,
                   rbf1_W, rbf1_b, rbf2_W, rbf2_b, ee_W, ee_b):
    W1 = ee_W[:H]
    W2 = ee_W[H:]
    relW1 = jnp.zeros((8, H), jnp.float32).at[:rel_emb.shape[0]].set(rel_emb) @ W1
    intW2 = int_emb @ W2
    m1 = jnp.zeros((32, H), jnp.float32).at[:20].set(rbf1_W) @ W2
    m2 = jnp.zeros((32, H), jnp.float32).at[:20].set(rbf2_W) @ W2
    c0 = (ee_b)[None, :]
    c1 = (rbf1_b @ W2 + ee_b)[None, :]
    c2 = (rbf2_b @ W2 + ee_b)[None, :]
    nb = E // _EB
    ew3 = edge_weight.reshape(E, 1)
    et3 = edge_type_list.reshape(E, 1)
    grid = (nb,)
    full = lambda s: pl.BlockSpec(s, lambda i: tuple(0 for _ in s))
    return pl.pallas_call(
        _edge_feat_body,
        grid=grid,
        in_specs=[
            pl.BlockSpec((_EB, 1), lambda i: (i, 0)),
            pl.BlockSpec((_EB, 1), lambda i: (i, 0)),
            full((8, H)), full((16, H)), full((32, H)), full((32, H)),
            full((1, H)), full((1, H)), full((1, H)),
        ],
        out_specs=pl.BlockSpec((_EB, H), lambda i: (i, 0)),
        out_shape=jax.ShapeDtypeStruct((E, H), jnp.float32),
    )(ew3, et3, relW1, intW2, m1, m2, c0, c1, c2)


def kernel(x, edge_index, edge_type, edge_type_list, edge_weight, rel_emb,
           int_emb, rbf1_W, rbf1_b, rbf2_W, rbf2_b, ee_W, ee_b,
           root0_W, root0_b, root1_W, root1_b, root2_W, root2_b,
           root3_W, root3_b, cat_W, cat_b):
    prop = edge_type_list == edge_type[0]
    sca = edge_type_list == edge_type[1]
    grp = edge_type_list == edge_type[2]
    x_edge = _edge_features(edge_weight, edge_type_list, rel_emb, int_emb,
                            rbf1_W, rbf1_b, rbf2_W, rbf2_b, ee_W, ee_b)
    src = edge_index[0]
    dst = edge_index[1]

    def conv(h, mask, Wr, br):
        m = (jnp.take(h, src, axis=0) + x_edge) * mask[:, None]
        ssum = jnp.zeros((N, H), h.dtype).at[dst].add(m)
        cnt = jnp.zeros((N,), h.dtype).at[dst].add(mask)
        agg = ssum / jnp.clip(cnt, 1.0)[:, None]
        msg = agg + h @ Wr + br
        return _leaky(_l2n(msg))

    mask_s = jnp.logical_or(prop, sca).astype(x.dtype)
    mask_g = jnp.logical_or(prop, grp).astype(x.dtype)
    roots = [(root0_W, root0_b), (root1_W, root1_b),
             (root2_W, root2_b), (root3_W, root3_b)]
    x_s = x
    x_g = x
    for i in range(2):
        x_s = conv(x_s, mask_s, roots[i][0], roots[i][1])
        x_g = conv(x_g, mask_g, roots[i + 2][0], roots[i + 2][1])
    x_mol = jnp.concatenate([x_s, x_g], axis=1) @ cat_W + cat_b
    return _leaky(x_mol)
